# baseline (speedup 1.0000x reference)
import functools

import jax
import jax.numpy as jnp
from jax import lax
from jax.experimental import pallas as pl
from jax.experimental.pallas import tpu as pltpu
from jax.experimental.pallas import tpu_sc as plsc

_NC = 2
_NS = 16
_NW = _NC * _NS
_LANES = 16
_MINOR = 128
_ROWQ = 32


def _rows_per_worker(total_bytes: int) -> int:
    per = -(-total_bytes // (_NW * _MINOR))
    return -(-per // _ROWQ) * _ROWQ


@functools.lru_cache(maxsize=None)
def _sc_mask_kernel(rows: int):
    total_rows = rows * _NW
    mesh = plsc.VectorSubcoreMesh(core_axis_name="c", subcore_axis_name="s")

    @functools.partial(
        pl.kernel,
        mesh=mesh,
        out_type=jax.ShapeDtypeStruct((total_rows, _MINOR), jnp.uint8),
        scratch_types=[
            pltpu.VMEM((rows, _MINOR), jnp.uint8),
            pltpu.VMEM((rows, _MINOR), jnp.uint8),
            pltpu.SemaphoreType.DMA,
            pltpu.SemaphoreType.DMA,
            pltpu.SemaphoreType.DMA,
        ],
    )
    def body(m_hbm, t_hbm, out_hbm, m_v, t_v, sem_m, sem_t, sem_o):
        wid = lax.axis_index("s") * _NC + lax.axis_index("c")
        base = wid * rows
        half = rows // 2

        def in_copies(h):
            lo = h * half
            cm = pltpu.make_async_copy(
                m_hbm.at[pl.ds(base + lo, half)], m_v.at[pl.ds(lo, half)], sem_m)
            ct = pltpu.make_async_copy(
                t_hbm.at[pl.ds(base + lo, half)], t_v.at[pl.ds(lo, half)], sem_t)
            return cm, ct

        cm0, ct0 = in_copies(0)
        cm1, ct1 = in_copies(1)
        cm0.start()
        ct0.start()
        cm1.start()
        ct1.start()

        mw = m_v.bitcast(jnp.int32)
        tw = t_v.bitcast(jnp.int32)

        def step(r, carry):
            for c in range(_MINOR // _LANES):
                sl = pl.ds(c * _LANES, _LANES)
                mw[r, sl] = mw[r, sl] & ~tw[r, sl]
            return carry

        out_copies = []
        for h in range(2):
            lo = h * half
            cm, ct = (cm0, ct0) if h == 0 else (cm1, ct1)
            cm.wait()
            ct.wait()
            lax.fori_loop(lo // 4, (lo + half) // 4, step, 0)
            co = pltpu.make_async_copy(
                m_v.at[pl.ds(lo, half)], out_hbm.at[pl.ds(base + lo, half)], sem_o)
            co.start()
            out_copies.append(co)
        for co in out_copies:
            co.wait()

    return body


def kernel(s0, s1, s2, mask, track_mask):
    n = mask.shape[0]
    rows = _rows_per_worker(n)
    total = rows * _NW * _MINOR

    m = jnp.pad(mask.view(jnp.uint8), (0, total - n)).reshape(rows * _NW, _MINOR)
    t = jnp.pad(track_mask.view(jnp.uint8), (0, total - n)).reshape(rows * _NW, _MINOR)
    out = _sc_mask_kernel(rows)(m, t)
    return (s0, s1, s2, out.reshape(total)[:n].view(jnp.bool_))


# speedup vs baseline: 1.0051x; 1.0051x over previous
import functools

import jax
import jax.numpy as jnp
from jax import lax
from jax.experimental import pallas as pl
from jax.experimental.pallas import tpu as pltpu
from jax.experimental.pallas import tpu_sc as plsc

_NC = 1
_NS = 16
_NW = _NC * _NS
_LANES = 16
_MINOR = 128
_ROWQ = 32


def _rows_per_worker(total_bytes: int) -> int:
    per = -(-total_bytes // (_NW * _MINOR))
    return -(-per // _ROWQ) * _ROWQ


@functools.lru_cache(maxsize=None)
def _sc_mask_kernel(rows: int):
    total_rows = rows * _NW
    mesh = plsc.VectorSubcoreMesh(
        core_axis_name="c", subcore_axis_name="s", num_cores=_NC)

    @functools.partial(
        pl.kernel,
        mesh=mesh,
        out_type=jax.ShapeDtypeStruct((total_rows, _MINOR), jnp.uint8),
        scratch_types=[
            pltpu.VMEM((rows, _MINOR), jnp.uint8),
            pltpu.VMEM((rows, _MINOR), jnp.uint8),
            pltpu.SemaphoreType.DMA,
            pltpu.SemaphoreType.DMA,
            pltpu.SemaphoreType.DMA,
        ],
    )
    def body(m_hbm, t_hbm, out_hbm, m_v, t_v, sem_m, sem_t, sem_o):
        wid = lax.axis_index("s") * _NC + lax.axis_index("c")
        base = wid * rows
        half = rows // 2

        def in_copies(h):
            lo = h * half
            cm = pltpu.make_async_copy(
                m_hbm.at[pl.ds(base + lo, half)], m_v.at[pl.ds(lo, half)], sem_m)
            ct = pltpu.make_async_copy(
                t_hbm.at[pl.ds(base + lo, half)], t_v.at[pl.ds(lo, half)], sem_t)
            return cm, ct

        cm0, ct0 = in_copies(0)
        cm1, ct1 = in_copies(1)
        cm0.start()
        ct0.start()
        cm1.start()
        ct1.start()

        mw = m_v.bitcast(jnp.int32)
        tw = t_v.bitcast(jnp.int32)

        def step(r, carry):
            for c in range(_MINOR // _LANES):
                sl = pl.ds(c * _LANES, _LANES)
                mw[r, sl] = mw[r, sl] & ~tw[r, sl]
            return carry

        out_copies = []
        for h in range(2):
            lo = h * half
            cm, ct = (cm0, ct0) if h == 0 else (cm1, ct1)
            cm.wait()
            ct.wait()
            lax.fori_loop(lo // 4, (lo + half) // 4, step, 0)
            co = pltpu.make_async_copy(
                m_v.at[pl.ds(lo, half)], out_hbm.at[pl.ds(base + lo, half)], sem_o)
            co.start()
            out_copies.append(co)
        for co in out_copies:
            co.wait()

    return body


def kernel(s0, s1, s2, mask, track_mask):
    n = mask.shape[0]
    rows = _rows_per_worker(n)
    total = rows * _NW * _MINOR

    m = jnp.pad(mask.view(jnp.uint8), (0, total - n)).reshape(rows * _NW, _MINOR)
    t = jnp.pad(track_mask.view(jnp.uint8), (0, total - n)).reshape(rows * _NW, _MINOR)
    out = _sc_mask_kernel(rows)(m, t)
    return (s0, s1, s2, out.reshape(total)[:n].view(jnp.bool_))
